# Initial kernel scaffold; baseline (speedup 1.0000x reference)
import jax
import jax.numpy as jnp
from jax.experimental import pallas as pl


def kernel(logits, labels):
    raise NotImplementedError("write your pallas kernel here")


# speedup vs baseline: 21.0256x; 21.0256x over previous
import functools

import jax
import jax.numpy as jnp
from jax import lax
from jax.experimental import pallas as pl
from jax.experimental.pallas import tpu as pltpu
from jax.experimental.pallas import tpu_sc as plsc

B = 16
N = 384 * 384
BITS = 15
NB = 1 << BITS
SHIFT = 32 - BITS
HALF = 1 << (SHIFT - 1)
L = 16
CHUNK = 4096
NCHUNK = N // CHUNK
HIST = 2 * NB
NUP = (NB // 2) // L


def _image_body(img, logits_hbm, labels_hbm, out_hbm, lbuf, ybuf, hist, obuf):
    zeros = jnp.zeros((L,), jnp.float32)

    def zero_body(i, _):
        hist[pl.ds(i * L, L)] = zeros
        return 0

    lax.fori_loop(0, HIST // L, zero_body, 0)

    ones = jnp.ones((L,), jnp.float32)

    def chunk_body(k, _):
        pltpu.sync_copy(logits_hbm.at[img, pl.ds(k * CHUNK, CHUNK)], lbuf)
        pltpu.sync_copy(labels_hbm.at[img, pl.ds(k * CHUNK, CHUNK)], ybuf)

        def vec_body(i, _):
            l = lbuf[pl.ds(i * L, L)]
            y = ybuf[pl.ds(i * L, L)]
            yf = y.astype(jnp.float32)
            e = (1.0 + l) - 2.0 * (l * yf)
            bits = plsc.bitcast(e, jnp.uint32)
            neg = bits >> 31
            m = (jnp.uint32(0) - neg) | jnp.uint32(0x80000000)
            key = bits ^ m
            ksat = jnp.minimum(key, jnp.uint32(0xFFFEFFFF))
            b = (ksat + jnp.uint32(HALF)) >> SHIFT
            yu = plsc.bitcast(y, jnp.uint32)
            idx = plsc.bitcast(b | (yu << BITS), jnp.int32)
            plsc.addupdate_scatter(hist, [idx], ones)
            return 0

        lax.fori_loop(0, CHUNK // L, vec_body, 0)
        return 0

    lax.fori_loop(0, NCHUNK, chunk_body, 0)

    def gsum(i, acc):
        return acc + hist[pl.ds(NB + i * L, L)]

    gvec = lax.fori_loop(0, NB // L, gsum, zeros)
    G = jnp.sum(gvec)
    G2 = jnp.float32(N) - G

    iota = lax.iota(jnp.int32, L)

    def scan_body(j, carry):
        acc, cn, cp = carry
        base = NB - (j + 1) * L
        Pd = lax.rev(hist[pl.ds(NB + base, L)], (0,))
        Qd = lax.rev(hist[pl.ds(base, L)], (0,))
        ip = plsc.cumsum(Pd)
        iq = plsc.cumsum(Qd)
        pbar = cp + (ip - Pd)
        nbar = cn + (iq - Qd)
        one = jnp.full((L,), 1.0, jnp.float32)
        d1 = G + nbar
        num1 = (G - pbar) * Qd + Pd * d1
        diff1 = jnp.where(d1 == 0.0, one, num1 / (d1 * (d1 + Qd)))
        d2 = G2 + pbar
        num2 = (G2 - nbar) * Pd + Qd * d2
        diff2 = jnp.where(d2 == 0.0, one, num2 / (d2 * (d2 + Pd)))
        hvec = (NB - 1 - j * L) - iota
        center = plsc.bitcast(hvec, jnp.uint32) << SHIFT
        eh = plsc.bitcast(center ^ jnp.uint32(0x80000000), jnp.float32)
        relu = jnp.maximum(eh, 0.0)
        contrib = relu * (diff1 + diff2)
        cnt = Pd + Qd
        acc = acc + jnp.where(cnt > 0.0, contrib, zeros)
        return (acc, cn + jnp.sum(Qd), cp + jnp.sum(Pd))

    acc, _, _ = lax.fori_loop(
        0, NUP, scan_body, (zeros, jnp.float32(0.0), jnp.float32(0.0)))
    loss = jnp.sum(acc) * 0.5
    obuf[...] = jnp.full((L,), loss)
    pltpu.sync_copy(obuf, out_hbm.at[img])


def _make_kernel():
    mesh = plsc.VectorSubcoreMesh(
        core_axis_name="c", subcore_axis_name="s", num_cores=2,
        num_subcores=16)

    @functools.partial(
        pl.kernel,
        out_type=jax.ShapeDtypeStruct((B, L), jnp.float32),
        mesh=mesh,
        scratch_types=[
            pltpu.VMEM((CHUNK,), jnp.float32),
            pltpu.VMEM((CHUNK,), jnp.int32),
            pltpu.VMEM((HIST,), jnp.float32),
            pltpu.VMEM((L,), jnp.float32),
        ],
        compiler_params=pltpu.CompilerParams(needs_layout_passes=False),
    )
    def kern(logits_hbm, labels_hbm, out_hbm, lbuf, ybuf, hist, obuf):
        c = lax.axis_index("c")
        s = lax.axis_index("s")
        img = c * 8 + s

        @pl.when(s < 8)
        def _():
            _image_body(img, logits_hbm, labels_hbm, out_hbm, lbuf, ybuf,
                        hist, obuf)

    return kern


_kern = _make_kernel()


def kernel(logits, labels):
    lf = logits.reshape(B, N)
    yf = labels.reshape(B, N)
    out = _kern(lf, yf)
    return jnp.mean(out[:, 0])


# speedup vs baseline: 32.6389x; 1.5523x over previous
import functools

import jax
import jax.numpy as jnp
from jax import lax
from jax.experimental import pallas as pl
from jax.experimental.pallas import tpu as pltpu
from jax.experimental.pallas import tpu_sc as plsc

B = 16
N = 384 * 384
BITS = 15
NB = 1 << BITS
SHIFT = 32 - BITS
HALF = 1 << (SHIFT - 1)
L = 16
CHUNK = 4096
NCHUNK = N // CHUNK
HCH = NCHUNK // 2
HIST = 2 * NB
NUP = (NB // 2) // L
MCH = 4096
UNROLL = 4


def _hist_chunk(lb, yb, hist, ys):
    ones = jnp.ones((L,), jnp.float32)

    def vb(i, ysacc):
        for u in range(UNROLL):
            off = (i * UNROLL + u) * L
            l = lb[pl.ds(off, L)]
            y = yb[pl.ds(off, L)]
            yf = y.astype(jnp.float32)
            e = (1.0 + l) - 2.0 * (l * yf)
            bits = plsc.bitcast(e, jnp.uint32)
            neg = bits >> 31
            m = (jnp.uint32(0) - neg) | jnp.uint32(0x80000000)
            key = bits ^ m
            ksat = jnp.minimum(key, jnp.uint32(0xFFFEFFFF))
            bkt = (ksat + jnp.uint32(HALF)) >> SHIFT
            yu = plsc.bitcast(y, jnp.uint32)
            idx = plsc.bitcast(bkt | (yu << BITS), jnp.int32)
            plsc.addupdate_scatter(hist, [idx], ones)
            ysacc = ysacc + yf
        return ysacc

    return lax.fori_loop(0, CHUNK // L // UNROLL, vb, ys)


def _make_kernel():
    mesh = plsc.VectorSubcoreMesh(
        core_axis_name="c", subcore_axis_name="s", num_cores=2,
        num_subcores=16)

    @functools.partial(
        pl.kernel,
        out_type=jax.ShapeDtypeStruct((B, L), jnp.float32),
        mesh=mesh,
        scratch_types=[
            pltpu.VMEM((CHUNK,), jnp.float32),
            pltpu.VMEM((CHUNK,), jnp.float32),
            pltpu.VMEM((CHUNK,), jnp.int32),
            pltpu.VMEM((CHUNK,), jnp.int32),
            pltpu.VMEM((HIST,), jnp.float32),
            pltpu.VMEM((MCH,), jnp.float32),
            pltpu.VMEM((L,), jnp.float32),
            pltpu.VMEM((L,), jnp.float32),
            pltpu.VMEM_SHARED((8, HIST), jnp.float32),
            pltpu.VMEM_SHARED((16, L), jnp.float32),
            pltpu.SemaphoreType.DMA,
            pltpu.SemaphoreType.DMA,
            pltpu.SemaphoreType.DMA,
            pltpu.SemaphoreType.DMA,
        ],
        compiler_params=pltpu.CompilerParams(needs_layout_passes=False),
    )
    def kern(logits_hbm, labels_hbm, out_hbm, lbuf0, lbuf1, ybuf0, ybuf1,
             hist, mbuf, ysp, obuf, sh_hist, sh_ys, sl0, sy0, sl1, sy1):
        c = lax.axis_index("c")
        s = lax.axis_index("s")
        slot = s % 8
        role = s // 8
        img = c * 8 + slot
        first = role * HCH

        zeros = jnp.zeros((L,), jnp.float32)

        def zero_body(i, _):
            hist[pl.ds(i * L, L)] = zeros
            return 0

        lax.fori_loop(0, HIST // L, zero_body, 0)

        def start(k, lb, yb, sl, sy):
            pltpu.async_copy(logits_hbm.at[img, pl.ds(k * CHUNK, CHUNK)],
                             lb, sl)
            pltpu.async_copy(labels_hbm.at[img, pl.ds(k * CHUNK, CHUNK)],
                             yb, sy)

        def wait(lb, yb, sl, sy, k):
            pltpu.make_async_copy(
                logits_hbm.at[img, pl.ds(k * CHUNK, CHUNK)], lb, sl).wait()
            pltpu.make_async_copy(
                labels_hbm.at[img, pl.ds(k * CHUNK, CHUNK)], yb, sy).wait()

        start(first, lbuf0, ybuf0, sl0, sy0)

        def body(jj, ys):
            k = first + jj * 2
            start(k + 1, lbuf1, ybuf1, sl1, sy1)
            wait(lbuf0, ybuf0, sl0, sy0, k)
            ys = _hist_chunk(lbuf0, ybuf0, hist, ys)

            @pl.when(jj < HCH // 2 - 1)
            def _():
                start(k + 2, lbuf0, ybuf0, sl0, sy0)

            wait(lbuf1, ybuf1, sl1, sy1, k + 1)
            ys = _hist_chunk(lbuf1, ybuf1, hist, ys)
            return ys

        ys = lax.fori_loop(0, HCH // 2, body, zeros)

        obuf[...] = ys
        pltpu.sync_copy(obuf, sh_ys.at[s])

        @pl.when(role == 1)
        def _():
            pltpu.sync_copy(hist, sh_hist.at[slot])

        plsc.subcore_barrier()

        @pl.when(role == 0)
        def _():
            def mbody(kk, _):
                pltpu.sync_copy(sh_hist.at[slot, pl.ds(kk * MCH, MCH)], mbuf)

                def madd(i, _):
                    off = i * L
                    hoff = kk * MCH + off
                    hist[pl.ds(hoff, L)] = (hist[pl.ds(hoff, L)]
                                            + mbuf[pl.ds(off, L)])
                    return 0

                lax.fori_loop(0, MCH // L, madd, 0)
                return 0

            lax.fori_loop(0, HIST // MCH, mbody, 0)

            pltpu.sync_copy(sh_ys.at[s + 8], ysp)
            G = jnp.sum(ys) + jnp.sum(ysp[...])
            G2 = jnp.float32(N) - G

            iota = lax.iota(jnp.int32, L)
            one = jnp.full((L,), 1.0, jnp.float32)

            def scan_body(j, carry):
                acc, cn, cp = carry
                base = NB - (j + 1) * L
                Pd = lax.rev(hist[pl.ds(NB + base, L)], (0,))
                Qd = lax.rev(hist[pl.ds(base, L)], (0,))
                ip = plsc.cumsum(Pd)
                iq = plsc.cumsum(Qd)
                pbar = cp + (ip - Pd)
                nbar = cn + (iq - Qd)
                d1 = G + nbar
                num1 = (G - pbar) * Qd + Pd * d1
                diff1 = jnp.where(d1 == 0.0, one, num1 / (d1 * (d1 + Qd)))
                d2 = G2 + pbar
                num2 = (G2 - nbar) * Pd + Qd * d2
                diff2 = jnp.where(d2 == 0.0, one, num2 / (d2 * (d2 + Pd)))
                hvec = (NB - 1 - j * L) - iota
                center = plsc.bitcast(hvec, jnp.uint32) << SHIFT
                eh = plsc.bitcast(center ^ jnp.uint32(0x80000000),
                                  jnp.float32)
                relu = jnp.maximum(eh, 0.0)
                contrib = relu * (diff1 + diff2)
                cnt = Pd + Qd
                acc = acc + jnp.where(cnt > 0.0, contrib, zeros)
                return (acc, cn + jnp.sum(Qd), cp + jnp.sum(Pd))

            acc, _, _ = lax.fori_loop(
                0, NUP, scan_body, (zeros, jnp.float32(0.0), jnp.float32(0.0)))
            loss = jnp.sum(acc) * 0.5
            obuf[...] = jnp.full((L,), loss)
            pltpu.sync_copy(obuf, out_hbm.at[img])

    return kern


_kern = _make_kernel()


def kernel(logits, labels):
    lf = logits.reshape(B, N)
    yf = labels.reshape(B, N)
    out = _kern(lf, yf)
    return jnp.mean(out[:, 0])


# speedup vs baseline: 38.0404x; 1.1655x over previous
import functools

import jax
import jax.numpy as jnp
from jax import lax
from jax.experimental import pallas as pl
from jax.experimental.pallas import tpu as pltpu
from jax.experimental.pallas import tpu_sc as plsc

B = 16
N = 384 * 384
BITS = 15
NB = 1 << BITS
SHIFT = 32 - BITS
HALF = 1 << (SHIFT - 1)
L = 16
CHUNK = 4096
NCHUNK = N // CHUNK
HCH = NCHUNK // 2
HIST = 2 * NB
NUP = (NB // 2) // L
MCH = 4096
UNROLL = 4


def _hist_chunk(lb, yb, hist, ys):
    ones = jnp.ones((L,), jnp.float32)

    def vb(i, ysacc):
        for u in range(UNROLL):
            off = (i * UNROLL + u) * L
            l = lb[pl.ds(off, L)]
            y = yb[pl.ds(off, L)]
            yf = y.astype(jnp.float32)
            e = (1.0 + l) - 2.0 * (l * yf)
            bits = plsc.bitcast(e, jnp.uint32)
            neg = bits >> 31
            m = (jnp.uint32(0) - neg) | jnp.uint32(0x80000000)
            key = bits ^ m
            ksat = jnp.minimum(key, jnp.uint32(0xFFFEFFFF))
            bkt = (ksat + jnp.uint32(HALF)) >> SHIFT
            yu = plsc.bitcast(y, jnp.uint32)
            idx = plsc.bitcast(bkt | (yu << BITS), jnp.int32)
            plsc.addupdate_scatter(hist, [idx], ones)
            ysacc = ysacc + yf
        return ysacc

    return lax.fori_loop(0, CHUNK // L // UNROLL, vb, ys)


def _make_kernel():
    mesh = plsc.VectorSubcoreMesh(
        core_axis_name="c", subcore_axis_name="s", num_cores=2,
        num_subcores=16)

    @functools.partial(
        pl.kernel,
        out_type=jax.ShapeDtypeStruct((B, L), jnp.float32),
        mesh=mesh,
        scratch_types=[
            pltpu.VMEM((CHUNK,), jnp.float32),
            pltpu.VMEM((CHUNK,), jnp.float32),
            pltpu.VMEM((CHUNK,), jnp.int32),
            pltpu.VMEM((CHUNK,), jnp.int32),
            pltpu.VMEM((HIST,), jnp.float32),
            pltpu.VMEM((MCH,), jnp.float32),
            pltpu.VMEM((L,), jnp.float32),
            pltpu.VMEM((L,), jnp.float32),
            pltpu.VMEM_SHARED((8, HIST), jnp.float32),
            pltpu.VMEM_SHARED((16, L), jnp.float32),
            pltpu.SemaphoreType.DMA,
            pltpu.SemaphoreType.DMA,
            pltpu.SemaphoreType.DMA,
            pltpu.SemaphoreType.DMA,
        ],
        compiler_params=pltpu.CompilerParams(needs_layout_passes=False),
    )
    def kern(logits_hbm, labels_hbm, out_hbm, lbuf0, lbuf1, ybuf0, ybuf1,
             hist, mbuf, ysp, obuf, sh_hist, sh_ys, sl0, sy0, sl1, sy1):
        c = lax.axis_index("c")
        s = lax.axis_index("s")
        slot = s % 8
        role = s // 8
        img = c * 8 + slot
        first = role * HCH

        zeros = jnp.zeros((L,), jnp.float32)

        def zero_body(i, _):
            for u in range(8):
                hist[pl.ds((i * 8 + u) * L, L)] = zeros
            return 0

        lax.fori_loop(0, HIST // L // 8, zero_body, 0)

        def start(k, lb, yb, sl, sy):
            pltpu.async_copy(logits_hbm.at[img, pl.ds(k * CHUNK, CHUNK)],
                             lb, sl)
            pltpu.async_copy(labels_hbm.at[img, pl.ds(k * CHUNK, CHUNK)],
                             yb, sy)

        def wait(lb, yb, sl, sy, k):
            pltpu.make_async_copy(
                logits_hbm.at[img, pl.ds(k * CHUNK, CHUNK)], lb, sl).wait()
            pltpu.make_async_copy(
                labels_hbm.at[img, pl.ds(k * CHUNK, CHUNK)], yb, sy).wait()

        start(first, lbuf0, ybuf0, sl0, sy0)

        def body(jj, ys):
            k = first + jj * 2
            start(k + 1, lbuf1, ybuf1, sl1, sy1)
            wait(lbuf0, ybuf0, sl0, sy0, k)
            ys = _hist_chunk(lbuf0, ybuf0, hist, ys)

            @pl.when(jj < HCH // 2 - 1)
            def _():
                start(k + 2, lbuf0, ybuf0, sl0, sy0)

            wait(lbuf1, ybuf1, sl1, sy1, k + 1)
            ys = _hist_chunk(lbuf1, ybuf1, hist, ys)
            return ys

        ys = lax.fori_loop(0, HCH // 2, body, zeros)

        obuf[...] = ys
        pltpu.sync_copy(obuf, sh_ys.at[s])

        @pl.when(role == 1)
        def _():
            pltpu.sync_copy(hist, sh_hist.at[slot])

        plsc.subcore_barrier()

        @pl.when(role == 0)
        def _():
            def mbody(kk, _):
                pltpu.sync_copy(sh_hist.at[slot, pl.ds(kk * MCH, MCH)], mbuf)

                def madd(i, _):
                    for u in range(4):
                        off = (i * 4 + u) * L
                        hoff = kk * MCH + off
                        hist[pl.ds(hoff, L)] = (hist[pl.ds(hoff, L)]
                                                + mbuf[pl.ds(off, L)])
                    return 0

                lax.fori_loop(0, MCH // L // 4, madd, 0)
                return 0

            lax.fori_loop(0, HIST // MCH, mbody, 0)

            pltpu.sync_copy(sh_ys.at[s + 8], ysp)
            G = jnp.sum(ys) + jnp.sum(ysp[...])
            G2 = jnp.float32(N) - G

            iota = lax.iota(jnp.int32, L)
            one = jnp.full((L,), 1.0, jnp.float32)
            lane15 = jnp.full((L,), L - 1, jnp.int32)

            def scan_step(j, acc, cnv, cpv):
                base = NB - (j + 1) * L
                Pd = lax.rev(hist[pl.ds(NB + base, L)], (0,))
                Qd = lax.rev(hist[pl.ds(base, L)], (0,))
                ip = plsc.cumsum(Pd)
                iq = plsc.cumsum(Qd)
                pbar = cpv + (ip - Pd)
                nbar = cnv + (iq - Qd)
                d1 = G + nbar
                num1 = (G - pbar) * Qd + Pd * d1
                diff1 = jnp.where(d1 == 0.0, one, num1 / (d1 * (d1 + Qd)))
                d2 = G2 + pbar
                num2 = (G2 - nbar) * Pd + Qd * d2
                diff2 = jnp.where(d2 == 0.0, one, num2 / (d2 * (d2 + Pd)))
                hvec = (NB - 1 - j * L) - iota
                center = plsc.bitcast(hvec, jnp.uint32) << SHIFT
                eh = plsc.bitcast(center ^ jnp.uint32(0x80000000),
                                  jnp.float32)
                relu = jnp.maximum(eh, 0.0)
                contrib = relu * (diff1 + diff2)
                cnt = Pd + Qd
                acc = acc + jnp.where(cnt > 0.0, contrib, zeros)
                cnv = cnv + jnp.take_along_axis(iq, lane15, axis=0)
                cpv = cpv + jnp.take_along_axis(ip, lane15, axis=0)
                return acc, cnv, cpv

            def scan_body(jj, carry):
                acc, cnv, cpv = carry
                acc, cnv, cpv = scan_step(jj * 2, acc, cnv, cpv)
                acc, cnv, cpv = scan_step(jj * 2 + 1, acc, cnv, cpv)
                return (acc, cnv, cpv)

            acc, _, _ = lax.fori_loop(
                0, NUP // 2, scan_body, (zeros, zeros, zeros))
            loss = jnp.sum(acc) * 0.5
            obuf[...] = jnp.full((L,), loss)
            pltpu.sync_copy(obuf, out_hbm.at[img])

    return kern


_kern = _make_kernel()


def kernel(logits, labels):
    lf = logits.reshape(B, N)
    yf = labels.reshape(B, N)
    out = _kern(lf, yf)
    return jnp.mean(out[:, 0])


# speedup vs baseline: 38.0999x; 1.0016x over previous
import functools

import jax
import jax.numpy as jnp
from jax import lax
from jax.experimental import pallas as pl
from jax.experimental.pallas import tpu as pltpu
from jax.experimental.pallas import tpu_sc as plsc

B = 16
N = 384 * 384
BITS = 15
NB = 1 << BITS
SHIFT = 32 - BITS
HALF = 1 << (SHIFT - 1)
L = 16
CHUNK = 4096
NCHUNK = N // CHUNK
HCH = NCHUNK // 2
HIST = 2 * NB
NUP = (NB // 2) // L
MCH = 4096
UNROLL = 4


def _hist_chunk(lb, yb, hist, ys):
    ones = jnp.ones((L,), jnp.float32)

    def vb(i, ysacc):
        for u in range(UNROLL):
            off = (i * UNROLL + u) * L
            l = lb[pl.ds(off, L)]
            y = yb[pl.ds(off, L)]
            yf = y.astype(jnp.float32)
            e = (1.0 + l) - 2.0 * (l * yf)
            bits = plsc.bitcast(e, jnp.uint32)
            neg = bits >> 31
            m = (jnp.uint32(0) - neg) | jnp.uint32(0x80000000)
            key = bits ^ m
            ksat = jnp.minimum(key, jnp.uint32(0xFFFEFFFF))
            bkt = (ksat + jnp.uint32(HALF)) >> SHIFT
            yu = plsc.bitcast(y, jnp.uint32)
            idx = plsc.bitcast(bkt | (yu << BITS), jnp.int32)
            plsc.addupdate_scatter(hist, [idx], ones)
            ysacc = ysacc + yf
        return ysacc

    return lax.fori_loop(0, CHUNK // L // UNROLL, vb, ys)


def _make_kernel():
    mesh = plsc.VectorSubcoreMesh(
        core_axis_name="c", subcore_axis_name="s", num_cores=2,
        num_subcores=16)

    @functools.partial(
        pl.kernel,
        out_type=jax.ShapeDtypeStruct((B, L), jnp.float32),
        mesh=mesh,
        scratch_types=[
            pltpu.VMEM((CHUNK,), jnp.float32),
            pltpu.VMEM((CHUNK,), jnp.float32),
            pltpu.VMEM((CHUNK,), jnp.int32),
            pltpu.VMEM((CHUNK,), jnp.int32),
            pltpu.VMEM((HIST,), jnp.float32),
            pltpu.VMEM((MCH,), jnp.float32),
            pltpu.VMEM((L,), jnp.float32),
            pltpu.VMEM((L,), jnp.float32),
            pltpu.VMEM_SHARED((8, HIST), jnp.float32),
            pltpu.VMEM_SHARED((16, L), jnp.float32),
            pltpu.SemaphoreType.DMA,
            pltpu.SemaphoreType.DMA,
            pltpu.SemaphoreType.DMA,
            pltpu.SemaphoreType.DMA,
        ],
        compiler_params=pltpu.CompilerParams(needs_layout_passes=False),
    )
    def kern(logits_hbm, labels_hbm, out_hbm, lbuf0, lbuf1, ybuf0, ybuf1,
             hist, mbuf, ysp, obuf, sh_hist, sh_ys, sl0, sy0, sl1, sy1):
        c = lax.axis_index("c")
        s = lax.axis_index("s")
        slot = s % 8
        role = s // 8
        img = c * 8 + slot
        first = role * HCH

        zeros = jnp.zeros((L,), jnp.float32)

        def zero_body(i, _):
            for u in range(8):
                hist[pl.ds((i * 8 + u) * L, L)] = zeros
            return 0

        lax.fori_loop(0, HIST // L // 8, zero_body, 0)

        def start(k, lb, yb, sl, sy):
            off = img * N + k * CHUNK
            pltpu.async_copy(logits_hbm.at[pl.ds(off, CHUNK)], lb, sl)
            pltpu.async_copy(labels_hbm.at[pl.ds(off, CHUNK)], yb, sy)

        def wait(lb, yb, sl, sy, k):
            off = img * N + k * CHUNK
            pltpu.make_async_copy(
                logits_hbm.at[pl.ds(off, CHUNK)], lb, sl).wait()
            pltpu.make_async_copy(
                labels_hbm.at[pl.ds(off, CHUNK)], yb, sy).wait()

        start(first, lbuf0, ybuf0, sl0, sy0)

        def body(jj, ys):
            k = first + jj * 2
            start(k + 1, lbuf1, ybuf1, sl1, sy1)
            wait(lbuf0, ybuf0, sl0, sy0, k)
            ys = _hist_chunk(lbuf0, ybuf0, hist, ys)

            @pl.when(jj < HCH // 2 - 1)
            def _():
                start(k + 2, lbuf0, ybuf0, sl0, sy0)

            wait(lbuf1, ybuf1, sl1, sy1, k + 1)
            ys = _hist_chunk(lbuf1, ybuf1, hist, ys)
            return ys

        ys = lax.fori_loop(0, HCH // 2, body, zeros)

        obuf[...] = ys
        pltpu.sync_copy(obuf, sh_ys.at[s])

        @pl.when(role == 1)
        def _():
            pltpu.sync_copy(hist, sh_hist.at[slot])

        plsc.subcore_barrier()

        @pl.when(role == 0)
        def _():
            def mbody(kk, _):
                pltpu.sync_copy(sh_hist.at[slot, pl.ds(kk * MCH, MCH)], mbuf)

                def madd(i, _):
                    for u in range(4):
                        off = (i * 4 + u) * L
                        hoff = kk * MCH + off
                        hist[pl.ds(hoff, L)] = (hist[pl.ds(hoff, L)]
                                                + mbuf[pl.ds(off, L)])
                    return 0

                lax.fori_loop(0, MCH // L // 4, madd, 0)
                return 0

            lax.fori_loop(0, HIST // MCH, mbody, 0)

            pltpu.sync_copy(sh_ys.at[s + 8], ysp)
            G = jnp.sum(ys) + jnp.sum(ysp[...])
            G2 = jnp.float32(N) - G

            iota = lax.iota(jnp.int32, L)
            one = jnp.full((L,), 1.0, jnp.float32)
            lane15 = jnp.full((L,), L - 1, jnp.int32)

            def scan_step(j, acc, cnv, cpv):
                base = NB - (j + 1) * L
                Pd = lax.rev(hist[pl.ds(NB + base, L)], (0,))
                Qd = lax.rev(hist[pl.ds(base, L)], (0,))
                ip = plsc.cumsum(Pd)
                iq = plsc.cumsum(Qd)
                pbar = cpv + (ip - Pd)
                nbar = cnv + (iq - Qd)
                d1 = G + nbar
                num1 = (G - pbar) * Qd + Pd * d1
                diff1 = jnp.where(d1 == 0.0, one, num1 / (d1 * (d1 + Qd)))
                d2 = G2 + pbar
                num2 = (G2 - nbar) * Pd + Qd * d2
                diff2 = jnp.where(d2 == 0.0, one, num2 / (d2 * (d2 + Pd)))
                hvec = (NB - 1 - j * L) - iota
                center = plsc.bitcast(hvec, jnp.uint32) << SHIFT
                eh = plsc.bitcast(center ^ jnp.uint32(0x80000000),
                                  jnp.float32)
                relu = jnp.maximum(eh, 0.0)
                contrib = relu * (diff1 + diff2)
                cnt = Pd + Qd
                acc = acc + jnp.where(cnt > 0.0, contrib, zeros)
                cnv = cnv + jnp.take_along_axis(iq, lane15, axis=0)
                cpv = cpv + jnp.take_along_axis(ip, lane15, axis=0)
                return acc, cnv, cpv

            def scan_body(jj, carry):
                acc, cnv, cpv = carry
                acc, cnv, cpv = scan_step(jj * 2, acc, cnv, cpv)
                acc, cnv, cpv = scan_step(jj * 2 + 1, acc, cnv, cpv)
                return (acc, cnv, cpv)

            acc, _, _ = lax.fori_loop(
                0, NUP // 2, scan_body, (zeros, zeros, zeros))
            loss = jnp.sum(acc) * 0.5
            obuf[...] = jnp.full((L,), loss)
            pltpu.sync_copy(obuf, out_hbm.at[img])

    return kern


_kern = _make_kernel()


def kernel(logits, labels):
    lf = logits.reshape(B * N)
    yf = labels.reshape(B * N)
    out = _kern(lf, yf)
    return jnp.mean(out[:, 0])
